# baseline (speedup 1.0000x reference)
import functools

import jax
import jax.numpy as jnp
from jax import lax
from jax.experimental import pallas as pl
from jax.experimental.pallas import tpu as pltpu
from jax.experimental.pallas import tpu_sc as plsc

NUM_EMBEDDINGS = 100000
EMBEDDING_DIM = 128
NUM_FEATURES = 26
BATCH_SIZE = 4096
TOTAL_ROWS = NUM_FEATURES * BATCH_SIZE

_INFO = plsc.get_sparse_core_info()
_NC = _INFO.num_cores
_NS = _INFO.num_subcores
_NW = _NC * _NS
CHUNK = BATCH_SIZE // _NW
NCHUNKS = NUM_FEATURES
NBUF = 3
_MAIN = (NCHUNKS // NBUF) * NBUF


def _gather_body(idx_hbm, table_hbm, out_hbm, idx_v, shared, *scratch):
    bufs = scratch[:NBUF]
    gsem = scratch[NBUF:2 * NBUF]
    csem = scratch[2 * NBUF:3 * NBUF]
    wsem = scratch[3 * NBUF:]
    sid = lax.axis_index("s")
    wid = sid * _NC + lax.axis_index("c")
    col = wid * CHUNK
    slots = tuple(shared.at[sid, b] for b in range(NBUF))

    pltpu.sync_copy(idx_hbm.at[:, pl.ds(col, CHUNK)], idx_v)

    def gather(f, b):
        return pltpu.async_copy(table_hbm.at[idx_v.at[f]], bufs[b], gsem[b])

    def gather_wait(f, b):
        pltpu.make_async_copy(
            table_hbm.at[idx_v.at[f]], bufs[b], gsem[b]).wait()

    def xbar(f, b):
        return pltpu.async_copy(bufs[b], slots[b], csem[b])

    def xbar_wait(f, b):
        pltpu.make_async_copy(bufs[b], slots[b], csem[b]).wait()

    def write(f, b):
        return pltpu.async_copy(
            slots[b], out_hbm.at[pl.ds(f * BATCH_SIZE + col, CHUNK)], wsem[b])

    def write_wait(f, b):
        pltpu.make_async_copy(
            slots[b], out_hbm.at[pl.ds(f * BATCH_SIZE + col, CHUNK)], wsem[b]
        ).wait()

    for b in range(NBUF - 1):
        gather(b, b)

    def step(f, b, traced):
        pb = (b + NBUF - 1) % NBUF
        nxt = f + NBUF - 1
        gather_wait(f, b)

        def _wait_w():
            write_wait(f - NBUF, b)

        if traced:
            pl.when(f - NBUF >= 0)(_wait_w)
        elif f - NBUF >= 0:
            _wait_w()
        xbar(f, b)

        def _flush_prev():
            xbar_wait(f - 1, pb)
            write(f - 1, pb)

        if traced:
            pl.when(f - 1 >= 0)(_flush_prev)
        elif f - 1 >= 0:
            _flush_prev()

        def _refill():
            gather(nxt, pb)

        if traced:
            pl.when(nxt < NCHUNKS)(_refill)
        elif nxt < NCHUNKS:
            _refill()

    def body(g, carry):
        for b in range(NBUF):
            step(NBUF * g + b, b, True)
        return carry

    lax.fori_loop(0, _MAIN // NBUF, body, 0)

    for f in range(_MAIN, NCHUNKS):
        step(f, f % NBUF, False)

    last = NCHUNKS - 1
    xbar_wait(last, last % NBUF)
    write(last, last % NBUF)
    for f in range(NCHUNKS - NBUF, NCHUNKS):
        write_wait(f, f % NBUF)


@jax.jit
def _gather(idx, table):
    mesh = plsc.VectorSubcoreMesh(core_axis_name="c", subcore_axis_name="s")
    k = functools.partial(
        pl.kernel,
        mesh=mesh,
        out_type=jax.ShapeDtypeStruct((TOTAL_ROWS, EMBEDDING_DIM), jnp.float32),
        scratch_types=[
            pltpu.VMEM((NCHUNKS, CHUNK), jnp.int32),
            pltpu.VMEM_SHARED((_NS, NBUF, CHUNK, EMBEDDING_DIM), jnp.float32),
        ]
        + [pltpu.VMEM((CHUNK, EMBEDDING_DIM), jnp.float32)] * NBUF
        + [pltpu.SemaphoreType.DMA] * (3 * NBUF),
    )(_gather_body)
    return k(idx, table)


def kernel(indices, lengths, weights):
    del lengths
    out = _gather(indices.astype(jnp.int32), weights)
    return out.reshape(NUM_FEATURES, BATCH_SIZE, EMBEDDING_DIM)


# speedup vs baseline: 1.0150x; 1.0150x over previous
import functools

import jax
import jax.numpy as jnp
from jax import lax
from jax.experimental import pallas as pl
from jax.experimental.pallas import tpu as pltpu
from jax.experimental.pallas import tpu_sc as plsc

NUM_EMBEDDINGS = 100000
EMBEDDING_DIM = 128
NUM_FEATURES = 26
BATCH_SIZE = 4096
TOTAL_ROWS = NUM_FEATURES * BATCH_SIZE

_INFO = plsc.get_sparse_core_info()
_NC = _INFO.num_cores
_NS = _INFO.num_subcores
_NW = _NC * _NS
CHUNK = BATCH_SIZE // _NW
NCHUNKS = NUM_FEATURES
NBUF = 7
_MAIN = (NCHUNKS // NBUF) * NBUF


def _gather_body(idx_hbm, table_hbm, out_hbm, idx_v, *scratch):
    bufs = scratch[:NBUF]
    gsem = scratch[NBUF:2 * NBUF]
    wsem = scratch[2 * NBUF:]
    wid = lax.axis_index("s") * _NC + lax.axis_index("c")
    col = wid * CHUNK

    pltpu.sync_copy(idx_hbm.at[:, pl.ds(col, CHUNK)], idx_v)

    def gather(f, b):
        return pltpu.async_copy(table_hbm.at[idx_v.at[f]], bufs[b], gsem[b])

    def gather_wait(f, b):
        pltpu.make_async_copy(
            table_hbm.at[idx_v.at[f]], bufs[b], gsem[b]).wait()

    def write(f, b):
        return pltpu.async_copy(
            bufs[b], out_hbm.at[pl.ds(f * BATCH_SIZE + col, CHUNK)], wsem[b])

    def write_wait(f, b):
        pltpu.make_async_copy(
            bufs[b], out_hbm.at[pl.ds(f * BATCH_SIZE + col, CHUNK)], wsem[b]
        ).wait()

    for b in range(NBUF - 1):
        gather(b, b)

    def body(g, carry):
        for b in range(NBUF):
            f = NBUF * g + b
            nb = (b + NBUF - 1) % NBUF
            nxt = f + NBUF - 1
            gather_wait(f, b)
            write(f, b)
            if b == 0:
                @pl.when(g > 0)
                def _():
                    write_wait(f - 1, nb)

                gather(nxt, nb)
            else:
                @pl.when(nxt < NCHUNKS)
                def _():
                    write_wait(f - 1, nb)
                    gather(nxt, nb)

        return carry

    lax.fori_loop(0, _MAIN // NBUF, body, 0)

    for f in range(_MAIN, NCHUNKS):
        gather_wait(f, f % NBUF)
        write(f, f % NBUF)

    for f in range(NCHUNKS - NBUF, NCHUNKS):
        write_wait(f, f % NBUF)


@jax.jit
def _gather(idx, table):
    mesh = plsc.VectorSubcoreMesh(core_axis_name="c", subcore_axis_name="s")
    k = functools.partial(
        pl.kernel,
        mesh=mesh,
        out_type=jax.ShapeDtypeStruct((TOTAL_ROWS, EMBEDDING_DIM), jnp.float32),
        scratch_types=[
            pltpu.VMEM((NCHUNKS, CHUNK), jnp.int32),
        ]
        + [pltpu.VMEM((CHUNK, EMBEDDING_DIM), jnp.float32)] * NBUF
        + [pltpu.SemaphoreType.DMA] * (2 * NBUF),
    )(_gather_body)
    return k(idx, table)


def kernel(indices, lengths, weights):
    del lengths
    out = _gather(indices.astype(jnp.int32), weights)
    return out.reshape(NUM_FEATURES, BATCH_SIZE, EMBEDDING_DIM)


# speedup vs baseline: 1.0202x; 1.0051x over previous
import functools

import jax
import jax.numpy as jnp
from jax import lax
from jax.experimental import pallas as pl
from jax.experimental.pallas import tpu as pltpu
from jax.experimental.pallas import tpu_sc as plsc

NUM_EMBEDDINGS = 100000
EMBEDDING_DIM = 128
NUM_FEATURES = 26
BATCH_SIZE = 4096
TOTAL_ROWS = NUM_FEATURES * BATCH_SIZE

_INFO = plsc.get_sparse_core_info()
_NC = _INFO.num_cores
_NS = _INFO.num_subcores
_NW = _NC * _NS
CHUNK = BATCH_SIZE // _NW
NCHUNKS = NUM_FEATURES
NBUF = 6
_MAIN = (NCHUNKS // NBUF) * NBUF


def _gather_body(idx_hbm, table_hbm, out_hbm, idx_v, *scratch):
    bufs = scratch[:NBUF]
    gsem = scratch[NBUF:2 * NBUF]
    wsem = scratch[2 * NBUF:]
    wid = lax.axis_index("s") * _NC + lax.axis_index("c")
    col = wid * CHUNK

    pltpu.sync_copy(idx_hbm.at[:, pl.ds(col, CHUNK)], idx_v)

    def gather(f, b):
        return pltpu.async_copy(table_hbm.at[idx_v.at[f]], bufs[b], gsem[b])

    def gather_wait(f, b):
        pltpu.make_async_copy(
            table_hbm.at[idx_v.at[f]], bufs[b], gsem[b]).wait()

    def write(f, b):
        return pltpu.async_copy(
            bufs[b], out_hbm.at[pl.ds(f * BATCH_SIZE + col, CHUNK)], wsem[b])

    def write_wait(f, b):
        pltpu.make_async_copy(
            bufs[b], out_hbm.at[pl.ds(f * BATCH_SIZE + col, CHUNK)], wsem[b]
        ).wait()

    for b in range(NBUF - 1):
        gather(b, b)

    def body(g, carry):
        for b in range(NBUF):
            f = NBUF * g + b
            nb = (b + NBUF - 1) % NBUF
            nxt = f + NBUF - 1
            gather_wait(f, b)
            write(f, b)
            if b == 0:
                @pl.when(g > 0)
                def _():
                    write_wait(f - 1, nb)

                gather(nxt, nb)
            else:
                @pl.when(nxt < NCHUNKS)
                def _():
                    write_wait(f - 1, nb)
                    gather(nxt, nb)

        return carry

    lax.fori_loop(0, _MAIN // NBUF, body, 0)

    for f in range(_MAIN, NCHUNKS):
        gather_wait(f, f % NBUF)
        write(f, f % NBUF)

    for f in range(NCHUNKS - NBUF, NCHUNKS):
        write_wait(f, f % NBUF)


@jax.jit
def _gather(idx, table):
    mesh = plsc.VectorSubcoreMesh(core_axis_name="c", subcore_axis_name="s")
    k = functools.partial(
        pl.kernel,
        mesh=mesh,
        out_type=jax.ShapeDtypeStruct((TOTAL_ROWS, EMBEDDING_DIM), jnp.float32),
        scratch_types=[
            pltpu.VMEM((NCHUNKS, CHUNK), jnp.int32),
        ]
        + [pltpu.VMEM((CHUNK, EMBEDDING_DIM), jnp.float32)] * NBUF
        + [pltpu.SemaphoreType.DMA] * (2 * NBUF),
    )(_gather_body)
    return k(idx, table)


def kernel(indices, lengths, weights):
    del lengths
    out = _gather(indices.astype(jnp.int32), weights)
    return out.reshape(NUM_FEATURES, BATCH_SIZE, EMBEDDING_DIM)
